# Initial kernel scaffold; baseline (speedup 1.0000x reference)
import jax
import jax.numpy as jnp
from jax.experimental import pallas as pl


def kernel(node_feats, edge_index, edge_feats, W_edge, b_edge, W_fc, attn_l, attn_r, bias):
    raise NotImplementedError("write your pallas kernel here")


# speedup vs baseline: 56.1654x; 56.1654x over previous
import functools

import jax
import jax.numpy as jnp
from jax import lax
from jax.experimental import pallas as pl
from jax.experimental.pallas import tpu as pltpu
from jax.experimental.pallas import tpu_sc as plsc

_DX = 144


def _prep_body(x_ref, wt_ref, at_ref, feat_ref, elr_ref, cmax_ref):
    i = pl.program_id(0)
    f = jnp.dot(x_ref[...], wt_ref[...], preferred_element_type=jnp.float32)
    feat_ref[:, :128] = f
    feat_ref[:, 128:] = jnp.zeros_like(feat_ref[:, 128:])
    elr = jnp.dot(f, at_ref[...], preferred_element_type=jnp.float32)
    elr_ref[...] = elr
    m = jnp.max(elr, axis=0, keepdims=True)

    @pl.when(i == 0)
    def _():
        cmax_ref[...] = m

    @pl.when(i != 0)
    def _():
        cmax_ref[...] = jnp.maximum(cmax_ref[...], m)


def _prep(node_feats, wfc_t, a_t, block_n):
    n, d_in = node_feats.shape
    d = wfc_t.shape[1]
    grid = n // block_n
    return pl.pallas_call(
        _prep_body,
        grid=(grid,),
        in_specs=[
            pl.BlockSpec((block_n, d_in), lambda i: (i, 0)),
            pl.BlockSpec((d_in, d), lambda i: (0, 0)),
            pl.BlockSpec((d, 16), lambda i: (0, 0)),
        ],
        out_specs=[
            pl.BlockSpec((block_n, _DX), lambda i: (i, 0)),
            pl.BlockSpec((block_n, 16), lambda i: (i, 0)),
            pl.BlockSpec((1, 16), lambda i: (0, 0)),
        ],
        out_shape=[
            jax.ShapeDtypeStruct((n, _DX), jnp.float32),
            jax.ShapeDtypeStruct((n, 16), jnp.float32),
            jax.ShapeDtypeStruct((1, 16), jnp.float32),
        ],
    )(node_feats, wfc_t, a_t)


def _make_edge_pass(n, e, h):
    nw = 32
    ns = 16
    e_per = e // nw
    n_full = e_per // 128
    tail = e_per - n_full * 128
    stripe = (n // ns) // 8 * 8
    stripe_last = n - (ns - 1) * stripe
    assert tail % 16 == 0 and stripe_last % 8 == 0 and nw * e_per == e

    mesh = plsc.VectorSubcoreMesh(core_axis_name="c", subcore_axis_name="s",
                                  num_cores=2, num_subcores=16)

    @functools.partial(
        pl.kernel,
        out_type=jax.ShapeDtypeStruct((2, n, _DX), jnp.float32),
        mesh=mesh,
        compiler_params=pltpu.CompilerParams(needs_layout_passes=False,
                                             use_tc_tiling_on_sc=False),
        scratch_types=[
            pltpu.VMEM_SHARED((n, _DX), jnp.float32),
            pltpu.VMEM((128, _DX), jnp.float32),
            pltpu.VMEM((128, 16), jnp.float32),
            pltpu.VMEM((128, 16), jnp.float32),
            pltpu.VMEM((128 * 16,), jnp.float32),
            pltpu.VMEM((128,), jnp.int32),
            pltpu.VMEM((128,), jnp.int32),
            pltpu.VMEM((16,), jnp.int32),
            pltpu.VMEM((16,), jnp.int32),
            pltpu.VMEM((16,), jnp.float32),
            pltpu.SemaphoreType.DMA,
        ],
    )
    def edge_pass(src_h, dst_h, feat_h, elr_h, cmax_h, accp_h,
                  accum_sh, rows_v, els_v, eld_v, wflat, srcv, dstv,
                  srcv_t, dstv_t, cvec, sem):
        c = lax.axis_index("c")
        s = lax.axis_index("s")
        zvec = jnp.zeros((16,), jnp.float32)
        iota16 = lax.iota(jnp.int32, 16)

        def _zrow(r, carry):
            for k in range(_DX // 16):
                rows_v[r, pl.ds(16 * k, 16)] = zvec
            wflat[pl.ds(r * 16, 16)] = zvec
            return carry

        lax.fori_loop(0, 128, _zrow, 0)

        row0 = pl.multiple_of(s * stripe, 8)

        def _zero_stripe(nrows):
            done = 0
            while done < nrows:
                step = min(128, nrows - done)
                pltpu.sync_copy(rows_v.at[pl.ds(0, step)],
                                accum_sh.at[pl.ds(row0 + done, step)])
                done += step

        @pl.when(s < ns - 1)
        def _():
            _zero_stripe(stripe)

        @pl.when(s == ns - 1)
        def _():
            _zero_stripe(stripe_last)

        pltpu.sync_copy(cmax_h.at[0], cvec)
        cvv = cvec[...]
        coff = []
        for hh in range(h):
            csum = cvv[hh] + cvv[4 + hh]
            coff.append(jnp.maximum(csum, 0.2 * csum))

        plsc.subcore_barrier()

        ebase = (c * ns + s) * e_per

        def _compute_w(es_ref, ed_ref, ngroups):
            for t in range(ngroups):
                rows16 = iota16 + 16 * t
                wdst = iota16 * 16 + (256 * t)
                for hh in range(h):
                    col_l = jnp.full((16,), hh, jnp.int32)
                    col_r = jnp.full((16,), 4 + hh, jnp.int32)
                    elg = plsc.load_gather(es_ref, [rows16, col_l])
                    erg = plsc.load_gather(ed_ref, [rows16, col_r])
                    sm = elg + erg
                    lr = jnp.maximum(sm, 0.2 * sm)
                    w = jnp.exp(lr - coff[hh])
                    plsc.store_scatter(wflat, [wdst + hh], w)

        def _scale_rows(nrows):
            def body(ei, carry):
                wrow = wflat[pl.ds(ei * 16, 16)]
                for hh in range(h):
                    wv = wrow[hh]
                    for k in range(2):
                        sl = pl.ds(32 * hh + 16 * k, 16)
                        rows_v[ei, sl] = rows_v[ei, sl] * wv
                rows_v[ei, pl.ds(128, 16)] = wrow
                return carry
            lax.fori_loop(0, nrows, body, 0)

        def chunk(i, carry):
            off = pl.multiple_of(ebase + i * 128, 8)
            pltpu.sync_copy(src_h.at[pl.ds(off, 128)], srcv)
            pltpu.sync_copy(dst_h.at[pl.ds(off, 128)], dstv)
            cp = pltpu.async_copy(feat_h.at[srcv], rows_v, sem)
            pltpu.sync_copy(elr_h.at[srcv], els_v)
            pltpu.sync_copy(elr_h.at[dstv], eld_v)
            _compute_w(els_v, eld_v, 8)
            cp.wait()
            _scale_rows(128)
            pltpu.sync_copy(rows_v, accum_sh.at[dstv], add=True)
            return carry

        lax.fori_loop(0, n_full, chunk, 0)

        if tail:
            off = pl.multiple_of(ebase + n_full * 128, 8)
            pltpu.sync_copy(src_h.at[pl.ds(off, tail)], srcv_t)
            pltpu.sync_copy(dst_h.at[pl.ds(off, tail)], dstv_t)
            cp = pltpu.async_copy(feat_h.at[srcv_t],
                                  rows_v.at[pl.ds(0, tail)], sem)
            pltpu.sync_copy(elr_h.at[srcv_t], els_v.at[pl.ds(0, tail)])
            pltpu.sync_copy(elr_h.at[dstv_t], eld_v.at[pl.ds(0, tail)])
            _compute_w(els_v, eld_v, tail // 16)
            cp.wait()
            _scale_rows(tail)
            pltpu.sync_copy(rows_v.at[pl.ds(0, tail)],
                            accum_sh.at[dstv_t], add=True)

        plsc.subcore_barrier()

        @pl.when(s < ns - 1)
        def _():
            pltpu.sync_copy(accum_sh.at[pl.ds(row0, stripe)],
                            accp_h.at[c, pl.ds(row0, stripe)])

        @pl.when(s == ns - 1)
        def _():
            pltpu.sync_copy(accum_sh.at[pl.ds(row0, stripe_last)],
                            accp_h.at[c, pl.ds(row0, stripe_last)])

    return edge_pass


def _fin_body(acc_ref, elr_ref, cmax_ref, feat_ref, bias_ref, e4_ref,
              out_ref):
    a = acc_ref[0] + acc_ref[1]
    elr = elr_ref[...]
    cm = cmax_ref[...]
    s_all = elr[:, :4] + elr[:, 4:8]
    c_all = cm[:, :4] + cm[:, 4:8]
    coff = jnp.maximum(c_all, 0.2 * c_all)
    sl = jnp.maximum(s_all, 0.2 * s_all)
    wself = jnp.exp(sl - coff)
    denom = a[:, 128:132] + wself
    e4 = e4_ref[...]
    wself_e = jnp.dot(wself, e4, preferred_element_type=jnp.float32)
    denom_e = jnp.dot(denom, e4, preferred_element_type=jnp.float32)
    out_ref[...] = ((a[:, :128] + wself_e * feat_ref[:, :128]) / denom_e
                    + bias_ref[...])


def _finalize(accp, elr, cmax, feat, bias2d, e4, block_n):
    n = feat.shape[0]
    d = 128
    grid = n // block_n
    return pl.pallas_call(
        _fin_body,
        grid=(grid,),
        in_specs=[
            pl.BlockSpec((2, block_n, _DX), lambda i: (0, i, 0)),
            pl.BlockSpec((block_n, 16), lambda i: (i, 0)),
            pl.BlockSpec((1, 16), lambda i: (0, 0)),
            pl.BlockSpec((block_n, _DX), lambda i: (i, 0)),
            pl.BlockSpec((1, d), lambda i: (0, 0)),
            pl.BlockSpec((4, d), lambda i: (0, 0)),
        ],
        out_specs=pl.BlockSpec((block_n, d), lambda i: (i, 0)),
        out_shape=jax.ShapeDtypeStruct((n, d), jnp.float32),
    )(accp, elr, cmax, feat, bias2d, e4)


def kernel(node_feats, edge_index, edge_feats, W_edge, b_edge, W_fc,
           attn_l, attn_r, bias):
    n, d_in = node_feats.shape
    e = edge_index.shape[1]
    h, d_out = attn_l.shape
    d = h * d_out

    a_t = jnp.zeros((d, 16), jnp.float32)
    e4 = jnp.zeros((h, d), jnp.float32)
    for hh in range(h):
        a_t = a_t.at[hh * d_out:(hh + 1) * d_out, hh].set(attn_l[hh])
        a_t = a_t.at[hh * d_out:(hh + 1) * d_out, h + hh].set(attn_r[hh])
        e4 = e4.at[hh, hh * d_out:(hh + 1) * d_out].set(1.0)

    feat, elr, cmax = _prep(node_feats, W_fc.T, a_t, block_n=1000)

    edge_pass = _make_edge_pass(n, e, h)
    accp = edge_pass(edge_index[0], edge_index[1], feat, elr, cmax)

    out = _finalize(accp, elr, cmax, feat, bias.reshape(1, d), e4,
                    block_n=1000)
    return out.reshape(n, h, d_out)


# speedup vs baseline: 70.0231x; 1.2467x over previous
import functools

import jax
import jax.numpy as jnp
from jax import lax
from jax.experimental import pallas as pl
from jax.experimental.pallas import tpu as pltpu
from jax.experimental.pallas import tpu_sc as plsc

_DX = 144


def _prep_body(x_ref, wt_ref, at_ref, feat_ref, elr_ref, cmax_ref):
    i = pl.program_id(0)
    f = jnp.dot(x_ref[...], wt_ref[...], preferred_element_type=jnp.float32)
    feat_ref[:, :128] = f
    feat_ref[:, 128:] = jnp.zeros_like(feat_ref[:, 128:])
    elr = jnp.dot(f, at_ref[...], preferred_element_type=jnp.float32)
    elr_ref[...] = elr
    m = jnp.max(elr, axis=0, keepdims=True)

    @pl.when(i == 0)
    def _():
        cmax_ref[...] = m

    @pl.when(i != 0)
    def _():
        cmax_ref[...] = jnp.maximum(cmax_ref[...], m)


def _prep(node_feats, wfc_t, a_t, block_n):
    n, d_in = node_feats.shape
    d = wfc_t.shape[1]
    grid = n // block_n
    return pl.pallas_call(
        _prep_body,
        grid=(grid,),
        in_specs=[
            pl.BlockSpec((block_n, d_in), lambda i: (i, 0)),
            pl.BlockSpec((d_in, d), lambda i: (0, 0)),
            pl.BlockSpec((d, 16), lambda i: (0, 0)),
        ],
        out_specs=[
            pl.BlockSpec((block_n, _DX), lambda i: (i, 0)),
            pl.BlockSpec((block_n, 16), lambda i: (i, 0)),
            pl.BlockSpec((1, 16), lambda i: (0, 0)),
        ],
        out_shape=[
            jax.ShapeDtypeStruct((n, _DX), jnp.float32),
            jax.ShapeDtypeStruct((n, 16), jnp.float32),
            jax.ShapeDtypeStruct((1, 16), jnp.float32),
        ],
    )(node_feats, wfc_t, a_t)


def _make_edge_pass(n, e, h):
    nw = 32
    ns = 16
    ch = 96
    e_per = e // nw
    n_full = e_per // ch
    tail = e_per - n_full * ch
    stripe = (n // ns) // 8 * 8
    stripe_last = n - (ns - 1) * stripe
    assert tail % 16 == 0 and stripe_last % 8 == 0 and nw * e_per == e
    assert n_full % 2 == 0

    mesh = plsc.VectorSubcoreMesh(core_axis_name="c", subcore_axis_name="s",
                                  num_cores=2, num_subcores=16)

    @functools.partial(
        pl.kernel,
        out_type=jax.ShapeDtypeStruct((2, n, _DX), jnp.float32),
        mesh=mesh,
        compiler_params=pltpu.CompilerParams(needs_layout_passes=False,
                                             use_tc_tiling_on_sc=False),
        scratch_types=[
            pltpu.VMEM_SHARED((n, _DX), jnp.float32),
            pltpu.VMEM((96, _DX), jnp.float32),
            pltpu.VMEM((96, _DX), jnp.float32),
            pltpu.VMEM((96, 16), jnp.float32),
            pltpu.VMEM((96, 16), jnp.float32),
            pltpu.VMEM((96, 16), jnp.float32),
            pltpu.VMEM((96, 16), jnp.float32),
            pltpu.VMEM((96 * 16,), jnp.float32),
            pltpu.VMEM((96,), jnp.int32),
            pltpu.VMEM((96,), jnp.int32),
            pltpu.VMEM((96,), jnp.int32),
            pltpu.VMEM((96,), jnp.int32),
            pltpu.VMEM((16,), jnp.int32),
            pltpu.VMEM((16,), jnp.int32),
            pltpu.VMEM((16,), jnp.float32),
            pltpu.SemaphoreType.DMA,
            pltpu.SemaphoreType.DMA,
            pltpu.SemaphoreType.DMA,
            pltpu.SemaphoreType.DMA,
            pltpu.SemaphoreType.DMA,
            pltpu.SemaphoreType.DMA,
            pltpu.SemaphoreType.DMA,
            pltpu.SemaphoreType.DMA,
        ],
    )
    def edge_pass(src_h, dst_h, feat_h, elr_h, cmax_h, accp_h,
                  accum_sh, rows_v, rows_v1, els_v, els_v1, eld_v, eld_v1,
                  wflat, srcv, srcv1, dstv, dstv1,
                  srcv_t, dstv_t, cvec,
                  semf0, semf1, seme0, seme1, semd0, semd1, sems0, sems1):
        rows_b = (rows_v, rows_v1)
        els_b = (els_v, els_v1)
        eld_b = (eld_v, eld_v1)
        srcv_b = (srcv, srcv1)
        dstv_b = (dstv, dstv1)
        semf_b = (semf0, semf1)
        seme_b = (seme0, seme1)
        semd_b = (semd0, semd1)
        sems_b = (sems0, sems1)
        c = lax.axis_index("c")
        s = lax.axis_index("s")
        zvec = jnp.zeros((16,), jnp.float32)
        iota16 = lax.iota(jnp.int32, 16)

        def _zrow(r, carry):
            for k in range(_DX // 16):
                rows_v[r, pl.ds(16 * k, 16)] = zvec
            wflat[pl.ds(r * 16, 16)] = zvec
            return carry

        lax.fori_loop(0, 96, _zrow, 0)

        row0 = pl.multiple_of(s * stripe, 8)

        def _zero_stripe(nrows):
            done = 0
            while done < nrows:
                step = min(96, nrows - done)
                pltpu.sync_copy(rows_v.at[pl.ds(0, step)],
                                accum_sh.at[pl.ds(row0 + done, step)])
                done += step

        @pl.when(s < ns - 1)
        def _():
            _zero_stripe(stripe)

        @pl.when(s == ns - 1)
        def _():
            _zero_stripe(stripe_last)

        pltpu.sync_copy(cmax_h.at[0], cvec)
        cvv = cvec[...]
        coff = []
        for hh in range(h):
            csum = cvv[hh] + cvv[4 + hh]
            coff.append(jnp.maximum(csum, 0.2 * csum))

        plsc.subcore_barrier()

        ebase = (c * ns + s) * e_per

        def _compute_w(es_ref, ed_ref, ngroups):
            for t in range(ngroups):
                rows16 = iota16 + 16 * t
                wdst = iota16 * 16 + (256 * t)
                for hh in range(h):
                    col_l = jnp.full((16,), hh, jnp.int32)
                    col_r = jnp.full((16,), 4 + hh, jnp.int32)
                    elg = plsc.load_gather(es_ref, [rows16, col_l])
                    erg = plsc.load_gather(ed_ref, [rows16, col_r])
                    sm = elg + erg
                    lr = jnp.maximum(sm, 0.2 * sm)
                    w = jnp.exp(lr - coff[hh])
                    plsc.store_scatter(wflat, [wdst + hh], w)

        def _scale_rows(rv, nrows):
            def body(ei, carry):
                wrow = wflat[pl.ds(ei * 16, 16)]
                for hh in range(h):
                    wv = wrow[hh]
                    for k in range(2):
                        sl = pl.ds(32 * hh + 16 * k, 16)
                        rv[ei, sl] = rv[ei, sl] * wv
                rv[ei, pl.ds(128, 16)] = wrow
                return carry
            lax.fori_loop(0, nrows, body, 0)

        def _issue(b, i):
            off = pl.multiple_of(ebase + i * ch, 8)
            pltpu.sync_copy(src_h.at[pl.ds(off, ch)], srcv_b[b])
            pltpu.sync_copy(dst_h.at[pl.ds(off, ch)], dstv_b[b])
            pltpu.async_copy(feat_h.at[srcv_b[b]], rows_b[b], semf_b[b])
            pltpu.async_copy(elr_h.at[srcv_b[b]], els_b[b], seme_b[b])
            pltpu.async_copy(elr_h.at[dstv_b[b]], eld_b[b], semd_b[b])

        def _wait_gathers(b):
            pltpu.make_async_copy(elr_h.at[srcv_b[b]], els_b[b],
                                  seme_b[b]).wait()
            pltpu.make_async_copy(elr_h.at[dstv_b[b]], eld_b[b],
                                  semd_b[b]).wait()

        def _wait_feat(b):
            pltpu.make_async_copy(feat_h.at[srcv_b[b]], rows_b[b],
                                  semf_b[b]).wait()

        def _wait_scatter(b):
            pltpu.make_async_copy(rows_b[b], accum_sh.at[dstv_b[b]],
                                  sems_b[b]).wait()

        def _process(b):
            _wait_gathers(b)
            _compute_w(els_b[b], eld_b[b], ch // 16)
            _wait_feat(b)
            _scale_rows(rows_b[b], ch)
            pltpu.async_copy(rows_b[b], accum_sh.at[dstv_b[b]], sems_b[b],
                             add=True)

        _issue(0, 0)

        def pair(j, carry):
            i0 = 2 * j
            _issue(1, i0 + 1)
            _process(0)

            @pl.when(i0 + 2 < n_full)
            def _():
                _wait_scatter(0)
                _issue(0, i0 + 2)

            _process(1)

            @pl.when(j + 1 < n_full // 2)
            def _():
                _wait_scatter(1)
            return carry

        lax.fori_loop(0, n_full // 2, pair, 0)
        _wait_scatter(0)
        _wait_scatter(1)

        if tail:
            off = pl.multiple_of(ebase + n_full * ch, 8)
            pltpu.sync_copy(src_h.at[pl.ds(off, tail)], srcv_t)
            pltpu.sync_copy(dst_h.at[pl.ds(off, tail)], dstv_t)
            cp = pltpu.async_copy(feat_h.at[srcv_t],
                                  rows_v.at[pl.ds(0, tail)], semf0)
            pltpu.sync_copy(elr_h.at[srcv_t], els_v.at[pl.ds(0, tail)])
            pltpu.sync_copy(elr_h.at[dstv_t], eld_v.at[pl.ds(0, tail)])
            _compute_w(els_v, eld_v, tail // 16)
            cp.wait()
            _scale_rows(rows_v, tail)
            pltpu.sync_copy(rows_v.at[pl.ds(0, tail)],
                            accum_sh.at[dstv_t], add=True)

        plsc.subcore_barrier()

        @pl.when(s < ns - 1)
        def _():
            pltpu.sync_copy(accum_sh.at[pl.ds(row0, stripe)],
                            accp_h.at[c, pl.ds(row0, stripe)])

        @pl.when(s == ns - 1)
        def _():
            pltpu.sync_copy(accum_sh.at[pl.ds(row0, stripe_last)],
                            accp_h.at[c, pl.ds(row0, stripe_last)])

    return edge_pass


def _fin_body(acc_ref, elr_ref, cmax_ref, feat_ref, bias_ref, e4_ref,
              out_ref):
    a = acc_ref[0] + acc_ref[1]
    elr = elr_ref[...]
    cm = cmax_ref[...]
    s_all = elr[:, :4] + elr[:, 4:8]
    c_all = cm[:, :4] + cm[:, 4:8]
    coff = jnp.maximum(c_all, 0.2 * c_all)
    sl = jnp.maximum(s_all, 0.2 * s_all)
    wself = jnp.exp(sl - coff)
    denom = a[:, 128:132] + wself
    e4 = e4_ref[...]
    wself_e = jnp.dot(wself, e4, preferred_element_type=jnp.float32)
    denom_e = jnp.dot(denom, e4, preferred_element_type=jnp.float32)
    out_ref[...] = ((a[:, :128] + wself_e * feat_ref[:, :128]) / denom_e
                    + bias_ref[...])


def _finalize(accp, elr, cmax, feat, bias2d, e4, block_n):
    n = feat.shape[0]
    d = 128
    grid = n // block_n
    return pl.pallas_call(
        _fin_body,
        grid=(grid,),
        in_specs=[
            pl.BlockSpec((2, block_n, _DX), lambda i: (0, i, 0)),
            pl.BlockSpec((block_n, 16), lambda i: (i, 0)),
            pl.BlockSpec((1, 16), lambda i: (0, 0)),
            pl.BlockSpec((block_n, _DX), lambda i: (i, 0)),
            pl.BlockSpec((1, d), lambda i: (0, 0)),
            pl.BlockSpec((4, d), lambda i: (0, 0)),
        ],
        out_specs=pl.BlockSpec((block_n, d), lambda i: (i, 0)),
        out_shape=jax.ShapeDtypeStruct((n, d), jnp.float32),
    )(accp, elr, cmax, feat, bias2d, e4)


def kernel(node_feats, edge_index, edge_feats, W_edge, b_edge, W_fc,
           attn_l, attn_r, bias):
    n, d_in = node_feats.shape
    e = edge_index.shape[1]
    h, d_out = attn_l.shape
    d = h * d_out

    a_t = jnp.zeros((d, 16), jnp.float32)
    e4 = jnp.zeros((h, d), jnp.float32)
    for hh in range(h):
        a_t = a_t.at[hh * d_out:(hh + 1) * d_out, hh].set(attn_l[hh])
        a_t = a_t.at[hh * d_out:(hh + 1) * d_out, h + hh].set(attn_r[hh])
        e4 = e4.at[hh, hh * d_out:(hh + 1) * d_out].set(1.0)

    feat, elr, cmax = _prep(node_feats, W_fc.T, a_t, block_n=1000)

    edge_pass = _make_edge_pass(n, e, h)
    accp = edge_pass(edge_index[0], edge_index[1], feat, elr, cmax)

    out = _finalize(accp, elr, cmax, feat, bias.reshape(1, d), e4,
                    block_n=1000)
    return out.reshape(n, h, d_out)


# speedup vs baseline: 79.1982x; 1.1310x over previous
import functools

import jax
import jax.numpy as jnp
from jax import lax
from jax.experimental import pallas as pl
from jax.experimental.pallas import tpu as pltpu
from jax.experimental.pallas import tpu_sc as plsc

_DX = 144


def _prep_body(x_ref, wt_ref, at_ref, feat_ref, elr_ref, cmax_ref):
    i = pl.program_id(0)
    f = jnp.dot(x_ref[...], wt_ref[...], preferred_element_type=jnp.float32)
    feat_ref[:, :128] = f
    feat_ref[:, 128:] = jnp.zeros_like(feat_ref[:, 128:])
    elr = jnp.dot(f, at_ref[...], preferred_element_type=jnp.float32)
    elr_ref[...] = elr
    m = jnp.max(elr, axis=0, keepdims=True)

    @pl.when(i == 0)
    def _():
        cmax_ref[...] = m

    @pl.when(i != 0)
    def _():
        cmax_ref[...] = jnp.maximum(cmax_ref[...], m)


def _prep(node_feats, wfc_t, a_t, block_n):
    n, d_in = node_feats.shape
    d = wfc_t.shape[1]
    grid = n // block_n
    return pl.pallas_call(
        _prep_body,
        grid=(grid,),
        in_specs=[
            pl.BlockSpec((block_n, d_in), lambda i: (i, 0)),
            pl.BlockSpec((d_in, d), lambda i: (0, 0)),
            pl.BlockSpec((d, 16), lambda i: (0, 0)),
        ],
        out_specs=[
            pl.BlockSpec((block_n, _DX), lambda i: (i, 0)),
            pl.BlockSpec((block_n, 16), lambda i: (i, 0)),
            pl.BlockSpec((1, 16), lambda i: (0, 0)),
        ],
        out_shape=[
            jax.ShapeDtypeStruct((n, _DX), jnp.float32),
            jax.ShapeDtypeStruct((n, 16), jnp.float32),
            jax.ShapeDtypeStruct((1, 16), jnp.float32),
        ],
    )(node_feats, wfc_t, a_t)


def _make_edge_pass(n, e, h):
    nw = 32
    ns = 16
    ch = 96
    e_per = e // nw
    n_full = e_per // ch
    tail = e_per - n_full * ch
    stripe = (n // ns) // 8 * 8
    stripe_last = n - (ns - 1) * stripe
    assert tail % 16 == 0 and stripe_last % 8 == 0 and nw * e_per == e
    assert n_full % 2 == 0

    mesh = plsc.VectorSubcoreMesh(core_axis_name="c", subcore_axis_name="s",
                                  num_cores=2, num_subcores=16)

    @functools.partial(
        pl.kernel,
        out_type=jax.ShapeDtypeStruct((2, n, _DX), jnp.float32),
        mesh=mesh,
        compiler_params=pltpu.CompilerParams(needs_layout_passes=False,
                                             use_tc_tiling_on_sc=False),
        scratch_types=[
            pltpu.VMEM_SHARED((n, _DX), jnp.float32),
            pltpu.VMEM((96, _DX), jnp.float32),
            pltpu.VMEM((96, _DX), jnp.float32),
            pltpu.VMEM((96, 16), jnp.float32),
            pltpu.VMEM((96, 16), jnp.float32),
            pltpu.VMEM((96, 16), jnp.float32),
            pltpu.VMEM((96, 16), jnp.float32),
            pltpu.VMEM((96 * 16,), jnp.float32),
            pltpu.VMEM((96,), jnp.int32),
            pltpu.VMEM((96,), jnp.int32),
            pltpu.VMEM((96,), jnp.int32),
            pltpu.VMEM((96,), jnp.int32),
            pltpu.VMEM((16,), jnp.int32),
            pltpu.VMEM((16,), jnp.int32),
            pltpu.VMEM((16,), jnp.float32),
            pltpu.SemaphoreType.DMA,
            pltpu.SemaphoreType.DMA,
            pltpu.SemaphoreType.DMA,
            pltpu.SemaphoreType.DMA,
            pltpu.SemaphoreType.DMA,
            pltpu.SemaphoreType.DMA,
            pltpu.SemaphoreType.DMA,
            pltpu.SemaphoreType.DMA,
        ],
    )
    def edge_pass(src_h, dst_h, feat_h, elr_h, cmax_h, accp_h,
                  accum_sh, rows_v, rows_v1, els_v, els_v1, eld_v, eld_v1,
                  wflat, srcv, srcv1, dstv, dstv1,
                  srcv_t, dstv_t, cvec,
                  semf0, semf1, seme0, seme1, semd0, semd1, sems0, sems1):
        rows_b = (rows_v, rows_v1)
        els_b = (els_v, els_v1)
        eld_b = (eld_v, eld_v1)
        srcv_b = (srcv, srcv1)
        dstv_b = (dstv, dstv1)
        semf_b = (semf0, semf1)
        seme_b = (seme0, seme1)
        semd_b = (semd0, semd1)
        sems_b = (sems0, sems1)
        c = lax.axis_index("c")
        s = lax.axis_index("s")
        zvec = jnp.zeros((16,), jnp.float32)
        iota16 = lax.iota(jnp.int32, 16)

        def _zrow(r, carry):
            for k in range(_DX // 16):
                rows_v[r, pl.ds(16 * k, 16)] = zvec
            wflat[pl.ds(r * 16, 16)] = zvec
            return carry

        lax.fori_loop(0, 96, _zrow, 0)

        row0 = pl.multiple_of(s * stripe, 8)

        def _zero_stripe(nrows):
            done = 0
            while done < nrows:
                step = min(96, nrows - done)
                pltpu.sync_copy(rows_v.at[pl.ds(0, step)],
                                accum_sh.at[pl.ds(row0 + done, step)])
                done += step

        @pl.when(s < ns - 1)
        def _():
            _zero_stripe(stripe)

        @pl.when(s == ns - 1)
        def _():
            _zero_stripe(stripe_last)

        pltpu.sync_copy(cmax_h.at[0], cvec)
        cvv = cvec[...]
        coff = []
        for hh in range(h):
            csum = cvv[hh] + cvv[4 + hh]
            coff.append(jnp.maximum(csum, 0.2 * csum))

        plsc.subcore_barrier()

        ebase = (c * ns + s) * e_per

        def _compute_w(es_ref, ed_ref, ngroups):
            for t in range(ngroups):
                rows16 = iota16 + 16 * t
                wdst = iota16 * 16 + (256 * t)
                for hh in range(h):
                    col_l = jnp.full((16,), hh, jnp.int32)
                    col_r = jnp.full((16,), 4 + hh, jnp.int32)
                    elg = plsc.load_gather(es_ref, [rows16, col_l])
                    erg = plsc.load_gather(ed_ref, [rows16, col_r])
                    sm = elg + erg
                    lr = jnp.maximum(sm, 0.2 * sm)
                    w = jnp.exp(lr - coff[hh])
                    plsc.store_scatter(wflat, [wdst + hh], w)

        def _scale_rows(rv, nrows):
            @plsc.parallel_loop(0, nrows, 1, unroll=4)
            def _(ei):
                wrow = wflat[pl.ds(ei * 16, 16)]
                for hh in range(h):
                    wv = wrow[hh]
                    for k in range(2):
                        sl = pl.ds(32 * hh + 16 * k, 16)
                        rv[ei, sl] = rv[ei, sl] * wv
                rv[ei, pl.ds(128, 16)] = wrow

        def _issue(b, i):
            off = pl.multiple_of(ebase + i * ch, 8)
            pltpu.sync_copy(src_h.at[pl.ds(off, ch)], srcv_b[b])
            pltpu.sync_copy(dst_h.at[pl.ds(off, ch)], dstv_b[b])
            pltpu.async_copy(feat_h.at[srcv_b[b]], rows_b[b], semf_b[b])
            pltpu.async_copy(elr_h.at[srcv_b[b]], els_b[b], seme_b[b])
            pltpu.async_copy(elr_h.at[dstv_b[b]], eld_b[b], semd_b[b])

        def _wait_gathers(b):
            pltpu.make_async_copy(elr_h.at[srcv_b[b]], els_b[b],
                                  seme_b[b]).wait()
            pltpu.make_async_copy(elr_h.at[dstv_b[b]], eld_b[b],
                                  semd_b[b]).wait()

        def _wait_feat(b):
            pltpu.make_async_copy(feat_h.at[srcv_b[b]], rows_b[b],
                                  semf_b[b]).wait()

        def _wait_scatter(b):
            pltpu.make_async_copy(rows_b[b], accum_sh.at[dstv_b[b]],
                                  sems_b[b]).wait()

        def _process(b):
            _wait_gathers(b)
            _compute_w(els_b[b], eld_b[b], ch // 16)
            _wait_feat(b)
            _scale_rows(rows_b[b], ch)
            pltpu.async_copy(rows_b[b], accum_sh.at[dstv_b[b]], sems_b[b],
                             add=True)

        _issue(0, 0)

        def pair(j, carry):
            i0 = 2 * j
            _issue(1, i0 + 1)
            _process(0)

            @pl.when(i0 + 2 < n_full)
            def _():
                _wait_scatter(0)
                _issue(0, i0 + 2)

            _process(1)

            @pl.when(j + 1 < n_full // 2)
            def _():
                _wait_scatter(1)
            return carry

        lax.fori_loop(0, n_full // 2, pair, 0)
        _wait_scatter(0)
        _wait_scatter(1)

        if tail:
            off = pl.multiple_of(ebase + n_full * ch, 8)
            pltpu.sync_copy(src_h.at[pl.ds(off, tail)], srcv_t)
            pltpu.sync_copy(dst_h.at[pl.ds(off, tail)], dstv_t)
            cp = pltpu.async_copy(feat_h.at[srcv_t],
                                  rows_v.at[pl.ds(0, tail)], semf0)
            pltpu.sync_copy(elr_h.at[srcv_t], els_v.at[pl.ds(0, tail)])
            pltpu.sync_copy(elr_h.at[dstv_t], eld_v.at[pl.ds(0, tail)])
            _compute_w(els_v, eld_v, tail // 16)
            cp.wait()
            _scale_rows(rows_v, tail)
            pltpu.sync_copy(rows_v.at[pl.ds(0, tail)],
                            accum_sh.at[dstv_t], add=True)

        plsc.subcore_barrier()

        @pl.when(s < ns - 1)
        def _():
            pltpu.sync_copy(accum_sh.at[pl.ds(row0, stripe)],
                            accp_h.at[c, pl.ds(row0, stripe)])

        @pl.when(s == ns - 1)
        def _():
            pltpu.sync_copy(accum_sh.at[pl.ds(row0, stripe_last)],
                            accp_h.at[c, pl.ds(row0, stripe_last)])

    return edge_pass


def _fin_body(acc_ref, elr_ref, cmax_ref, feat_ref, bias_ref, e4_ref,
              out_ref):
    a = acc_ref[0] + acc_ref[1]
    elr = elr_ref[...]
    cm = cmax_ref[...]
    s_all = elr[:, :4] + elr[:, 4:8]
    c_all = cm[:, :4] + cm[:, 4:8]
    coff = jnp.maximum(c_all, 0.2 * c_all)
    sl = jnp.maximum(s_all, 0.2 * s_all)
    wself = jnp.exp(sl - coff)
    denom = a[:, 128:132] + wself
    e4 = e4_ref[...]
    wself_e = jnp.dot(wself, e4, preferred_element_type=jnp.float32)
    denom_e = jnp.dot(denom, e4, preferred_element_type=jnp.float32)
    out_ref[...] = ((a[:, :128] + wself_e * feat_ref[:, :128]) / denom_e
                    + bias_ref[...])


def _finalize(accp, elr, cmax, feat, bias2d, e4, block_n):
    n = feat.shape[0]
    d = 128
    grid = n // block_n
    return pl.pallas_call(
        _fin_body,
        grid=(grid,),
        in_specs=[
            pl.BlockSpec((2, block_n, _DX), lambda i: (0, i, 0)),
            pl.BlockSpec((block_n, 16), lambda i: (i, 0)),
            pl.BlockSpec((1, 16), lambda i: (0, 0)),
            pl.BlockSpec((block_n, _DX), lambda i: (i, 0)),
            pl.BlockSpec((1, d), lambda i: (0, 0)),
            pl.BlockSpec((4, d), lambda i: (0, 0)),
        ],
        out_specs=pl.BlockSpec((block_n, d), lambda i: (i, 0)),
        out_shape=jax.ShapeDtypeStruct((n, d), jnp.float32),
    )(accp, elr, cmax, feat, bias2d, e4)


def kernel(node_feats, edge_index, edge_feats, W_edge, b_edge, W_fc,
           attn_l, attn_r, bias):
    n, d_in = node_feats.shape
    e = edge_index.shape[1]
    h, d_out = attn_l.shape
    d = h * d_out

    a_t = jnp.zeros((d, 16), jnp.float32)
    e4 = jnp.zeros((h, d), jnp.float32)
    for hh in range(h):
        a_t = a_t.at[hh * d_out:(hh + 1) * d_out, hh].set(attn_l[hh])
        a_t = a_t.at[hh * d_out:(hh + 1) * d_out, h + hh].set(attn_r[hh])
        e4 = e4.at[hh, hh * d_out:(hh + 1) * d_out].set(1.0)

    feat, elr, cmax = _prep(node_feats, W_fc.T, a_t, block_n=1000)

    edge_pass = _make_edge_pass(n, e, h)
    accp = edge_pass(edge_index[0], edge_index[1], feat, elr, cmax)

    out = _finalize(accp, elr, cmax, feat, bias.reshape(1, d), e4,
                    block_n=1000)
    return out.reshape(n, h, d_out)
